# baseline (speedup 1.0000x reference)
import functools

import jax
import jax.numpy as jnp
from jax import lax
from jax.experimental import pallas as pl
from jax.experimental.pallas import tpu as pltpu
from jax.experimental.pallas import tpu_sc as plsc

BATCH = 16384
HIST = 50
DIM = 32

NC = 2
NS = 16
NW = NC * NS
B = BATCH * HIST
SUB = 64
NSUB = 20
CHUNK = SUB * NSUB
NCHUNKS = B // (NW * CHUNK)

_mesh = plsc.VectorSubcoreMesh(core_axis_name="c", subcore_axis_name="s")


@functools.partial(
    pl.kernel,
    mesh=_mesh,
    out_type=jax.ShapeDtypeStruct((NW * NCHUNKS, NSUB, SUB, DIM), jnp.float32),
    scratch_types=[
        pltpu.VMEM((2, NSUB, SUB), jnp.int32),
        pltpu.VMEM((2, NSUB, SUB, DIM), jnp.float32),
        pltpu.SemaphoreType.DMA,
        pltpu.SemaphoreType.DMA,
        pltpu.SemaphoreType.DMA,
        pltpu.SemaphoreType.DMA,
        pltpu.SemaphoreType.DMA,
        pltpu.SemaphoreType.DMA,
    ],
    compiler_params=pltpu.CompilerParams(use_tc_tiling_on_sc=False),
)
def _sc_gather(idx_hbm, table_hbm, out_hbm, idx_v, rows_v,
               sem_g0, sem_g1, sem_i0, sem_i1, sem_o0, sem_o1):
    wid = lax.axis_index("s") * NC + lax.axis_index("c")
    base = wid * NCHUNKS
    sem_g = (sem_g0, sem_g1)
    sem_i = (sem_i0, sem_i1)
    sem_o = (sem_o0, sem_o1)

    def issue_gathers(p):
        for j in range(NSUB):
            pltpu.async_copy(table_hbm.at[idx_v.at[p].at[j]],
                             rows_v.at[p].at[j], sem_g[p])

    def wait_gathers(p):
        for j in range(NSUB):
            pltpu.make_async_copy(table_hbm.at[idx_v.at[p].at[j]],
                                  rows_v.at[p].at[j], sem_g[p]).wait()

    def issue_idx(c, p):
        pltpu.async_copy(idx_hbm.at[base + c], idx_v.at[p], sem_i[p])

    def wait_idx(p):
        pltpu.make_async_copy(idx_hbm.at[base], idx_v.at[p], sem_i[p]).wait()

    def issue_out(c, p):
        pltpu.async_copy(rows_v.at[p], out_hbm.at[base + c], sem_o[p])

    def wait_out(p):
        pltpu.make_async_copy(rows_v.at[p], out_hbm.at[base], sem_o[p]).wait()

    def half(c, p):
        q = 1 - p
        wait_idx(q)
        wait_out(q)
        issue_gathers(q)
        wait_gathers(p)
        issue_out(c, p)
        issue_idx(c + 2, p)

    pltpu.sync_copy(idx_hbm.at[base], idx_v.at[0])
    issue_gathers(0)
    issue_idx(1, 1)
    wait_idx(1)
    issue_gathers(1)
    wait_gathers(0)
    issue_out(0, 0)
    issue_idx(2, 0)

    def body2(i, carry):
        c = 2 * i + 1
        half(c, 1)
        half(c + 1, 0)
        return carry

    lax.fori_loop(0, (NCHUNKS - 4) // 2, body2, 0)
    half(NCHUNKS - 3, 1)

    wait_idx(1)
    wait_out(1)
    issue_gathers(1)
    wait_gathers(0)
    issue_out(NCHUNKS - 2, 0)
    wait_gathers(1)
    issue_out(NCHUNKS - 1, 1)
    wait_out(0)
    wait_out(1)


def kernel(sets, E):
    idx = sets.reshape(NW * NCHUNKS, NSUB, SUB)
    out = _sc_gather(idx, E)
    return out.reshape(BATCH, HIST, DIM)


# speedup vs baseline: 1.0005x; 1.0005x over previous
import functools

import jax
import jax.numpy as jnp
from jax import lax
from jax.experimental import pallas as pl
from jax.experimental.pallas import tpu as pltpu
from jax.experimental.pallas import tpu_sc as plsc

BATCH = 16384
HIST = 50
DIM = 32

NC = 2
NS = 16
NW = NC * NS
RPC = 16
ROWS_PER_W = BATCH // NW
NCHUNKS = ROWS_PER_W // RPC

_mesh = plsc.VectorSubcoreMesh(core_axis_name="c", subcore_axis_name="s")


@functools.partial(
    pl.kernel,
    mesh=_mesh,
    out_type=jax.ShapeDtypeStruct((BATCH, HIST, DIM), jnp.float32),
    scratch_types=[
        pltpu.VMEM((2, RPC, HIST), jnp.int32),
        pltpu.VMEM((2, RPC, HIST, DIM), jnp.float32),
        pltpu.SemaphoreType.DMA,
        pltpu.SemaphoreType.DMA,
        pltpu.SemaphoreType.DMA,
        pltpu.SemaphoreType.DMA,
        pltpu.SemaphoreType.DMA,
        pltpu.SemaphoreType.DMA,
    ],
    compiler_params=pltpu.CompilerParams(use_tc_tiling_on_sc=False),
)
def _sc_gather(idx_hbm, table_hbm, out_hbm, idx_v, rows_v,
               sem_g0, sem_g1, sem_i0, sem_i1, sem_o0, sem_o1):
    wid = lax.axis_index("s") * NC + lax.axis_index("c")
    base = wid * ROWS_PER_W
    sem_g = (sem_g0, sem_g1)
    sem_i = (sem_i0, sem_i1)
    sem_o = (sem_o0, sem_o1)

    def issue_gathers(p):
        for j in range(RPC):
            pltpu.async_copy(table_hbm.at[idx_v.at[p].at[j]],
                             rows_v.at[p].at[j], sem_g[p])

    def wait_gathers(p):
        for j in range(RPC):
            pltpu.make_async_copy(table_hbm.at[idx_v.at[p].at[j]],
                                  rows_v.at[p].at[j], sem_g[p]).wait()

    def issue_idx(c, p):
        pltpu.async_copy(idx_hbm.at[pl.ds(base + c * RPC, RPC)], idx_v.at[p],
                         sem_i[p])

    def wait_idx(p):
        pltpu.make_async_copy(idx_hbm.at[pl.ds(base, RPC)], idx_v.at[p],
                              sem_i[p]).wait()

    def issue_out(c, p):
        pltpu.async_copy(rows_v.at[p], out_hbm.at[pl.ds(base + c * RPC, RPC)],
                         sem_o[p])

    def wait_out(p):
        pltpu.make_async_copy(rows_v.at[p], out_hbm.at[pl.ds(base, RPC)],
                              sem_o[p]).wait()

    def half(c, p):
        q = 1 - p
        wait_idx(q)
        wait_out(q)
        issue_gathers(q)
        wait_gathers(p)
        issue_out(c, p)
        issue_idx(c + 2, p)

    pltpu.sync_copy(idx_hbm.at[pl.ds(base, RPC)], idx_v.at[0])
    issue_gathers(0)
    issue_idx(1, 1)
    wait_idx(1)
    issue_gathers(1)
    wait_gathers(0)
    issue_out(0, 0)
    issue_idx(2, 0)

    def body2(i, carry):
        c = 2 * i + 1
        half(c, 1)
        half(c + 1, 0)
        return carry

    lax.fori_loop(0, (NCHUNKS - 4) // 2, body2, 0)
    half(NCHUNKS - 3, 1)

    wait_idx(1)
    wait_out(1)
    issue_gathers(1)
    wait_gathers(0)
    issue_out(NCHUNKS - 2, 0)
    wait_gathers(1)
    issue_out(NCHUNKS - 1, 1)
    wait_out(0)
    wait_out(1)


def kernel(sets, E):
    return _sc_gather(sets, E)
